# Initial kernel scaffold; baseline (speedup 1.0000x reference)
import jax
import jax.numpy as jnp
from jax.experimental import pallas as pl


def kernel(atomic_numbers, batch_idx, freq_ratios):
    raise NotImplementedError("write your pallas kernel here")


# speedup vs baseline: 26.9426x; 26.9426x over previous
import functools

import jax
import jax.numpy as jnp
from jax import lax
from jax.experimental import pallas as pl
from jax.experimental.pallas import tpu as pltpu
from jax.experimental.pallas import tpu_sc as plsc

N = 100000
ATOM_TYPES = 119
NUM_GRAPHS = 1024

L = 16
NTILES = 16
CHUNK = 6256
NPAD = CHUNK * NTILES
NVEC = CHUNK // L
TBL = 128
SEG_PAD = 1152
SEG_BLK = 128

_mesh = plsc.VectorSubcoreMesh(
    core_axis_name="c", subcore_axis_name="s", num_cores=1)


@functools.partial(
    pl.kernel,
    out_type=jax.ShapeDtypeStruct((NPAD,), jnp.float32),
    mesh=_mesh,
    compiler_params=pltpu.CompilerParams(needs_layout_passes=False),
    scratch_types=[
        pltpu.VMEM((CHUNK,), jnp.int32),
        pltpu.VMEM((CHUNK,), jnp.int32),
        pltpu.VMEM((CHUNK,), jnp.float32),
        pltpu.VMEM((TBL,), jnp.float32),
        pltpu.VMEM((SEG_PAD,), jnp.float32),
        pltpu.VMEM((SEG_PAD,), jnp.float32),
        pltpu.VMEM((NTILES, SEG_BLK), jnp.float32),
        pltpu.VMEM((NTILES, SEG_BLK), jnp.float32),
        pltpu.VMEM((SEG_PAD,), jnp.float32),
        pltpu.VMEM_SHARED((NTILES, SEG_PAD), jnp.float32),
        pltpu.VMEM_SHARED((NTILES, SEG_PAD), jnp.float32),
        pltpu.VMEM_SHARED((SEG_PAD,), jnp.float32),
    ],
)
def _sc_multiplier(an_hbm, bid_hbm, fr_hbm, out_hbm,
                   an_v, bid_v, raw_v, tbl_v, acc_sum, acc_cnt,
                   red_sum, red_cnt, inv_v, sh_sum, sh_cnt, sh_inv):
    w = lax.axis_index("s")
    base = w * CHUNK

    pltpu.sync_copy(an_hbm.at[pl.ds(base, CHUNK)], an_v)
    pltpu.sync_copy(bid_hbm.at[pl.ds(base, CHUNK)], bid_v)
    pltpu.sync_copy(fr_hbm, tbl_v)

    eps = jnp.float32(1e-8)
    for j in range(TBL // L):
        f = tbl_v[pl.ds(j * L, L)]
        tbl_v[pl.ds(j * L, L)] = 1.0 / (f + eps)

    zero16 = jnp.zeros((L,), jnp.float32)

    def zero_body(j, carry):
        acc_sum[pl.ds(j * L, L)] = zero16
        acc_cnt[pl.ds(j * L, L)] = zero16
        return carry

    lax.fori_loop(0, SEG_PAD // L, zero_body, 0)

    ones16 = jnp.ones((L,), jnp.float32)

    def pass1(i, carry):
        a = an_v[pl.ds(i * L, L)]
        b = bid_v[pl.ds(i * L, L)]
        r = plsc.load_gather(tbl_v, [a])
        raw_v[pl.ds(i * L, L)] = r
        plsc.addupdate_scatter(acc_sum, [b], r)
        plsc.addupdate_scatter(acc_cnt, [b], ones16)
        return carry

    lax.fori_loop(0, NVEC, pass1, 0)

    pltpu.sync_copy(acc_sum, sh_sum.at[w])
    pltpu.sync_copy(acc_cnt, sh_cnt.at[w])
    plsc.subcore_barrier()

    @pl.when(w < 8)
    def _():
        seg0 = w * SEG_BLK
        pltpu.sync_copy(sh_sum.at[:, pl.ds(seg0, SEG_BLK)], red_sum)
        pltpu.sync_copy(sh_cnt.at[:, pl.ds(seg0, SEG_BLK)], red_cnt)
        for sub in range(SEG_BLK // L):
            s = jnp.zeros((L,), jnp.float32)
            c = jnp.zeros((L,), jnp.float32)
            for t in range(NTILES):
                s = s + red_sum[t, pl.ds(sub * L, L)]
                c = c + red_cnt[t, pl.ds(sub * L, L)]
            inv_v[pl.ds(sub * L, L)] = jnp.maximum(c, 1.0) / s
        pltpu.sync_copy(inv_v.at[pl.ds(0, SEG_BLK)],
                        sh_inv.at[pl.ds(seg0, SEG_BLK)])

    @pl.when(w == 8)
    def _():
        for sub in range(SEG_BLK // L):
            inv_v[pl.ds(sub * L, L)] = ones16
        pltpu.sync_copy(inv_v.at[pl.ds(0, SEG_BLK)],
                        sh_inv.at[pl.ds(NUM_GRAPHS, SEG_BLK)])

    plsc.subcore_barrier()
    pltpu.sync_copy(sh_inv, inv_v)

    def pass2(i, carry):
        b = bid_v[pl.ds(i * L, L)]
        r = raw_v[pl.ds(i * L, L)]
        m = plsc.load_gather(inv_v, [b])
        raw_v[pl.ds(i * L, L)] = r * m
        return carry

    lax.fori_loop(0, NVEC, pass2, 0)
    pltpu.sync_copy(raw_v, out_hbm.at[pl.ds(base, CHUNK)])


def kernel(atomic_numbers, batch_idx, freq_ratios):
    an = jnp.pad(atomic_numbers.astype(jnp.int32), (0, NPAD - N))
    bid = jnp.pad(batch_idx.astype(jnp.int32), (0, NPAD - N),
                  constant_values=NUM_GRAPHS)
    fr = jnp.pad(freq_ratios, (0, TBL - ATOM_TYPES))
    out = _sc_multiplier(an, bid, fr)
    return out[:N]


# speedup vs baseline: 30.4085x; 1.1286x over previous
import functools

import jax
import jax.numpy as jnp
from jax import lax
from jax.experimental import pallas as pl
from jax.experimental.pallas import tpu as pltpu
from jax.experimental.pallas import tpu_sc as plsc

N = 100000
ATOM_TYPES = 119
NUM_GRAPHS = 1024

L = 16
NTILES = 16
CH_MAIN = 6256
CH_TAIL = N - 15 * CH_MAIN
TBL = 128
SEG_PAD = NUM_GRAPHS
SEG_BLK = 128
UNROLL = 4

_mesh = plsc.VectorSubcoreMesh(
    core_axis_name="c", subcore_axis_name="s", num_cores=1)


@functools.partial(
    pl.kernel,
    out_type=jax.ShapeDtypeStruct((N,), jnp.float32),
    mesh=_mesh,
    compiler_params=pltpu.CompilerParams(needs_layout_passes=False),
    scratch_types=[
        pltpu.VMEM((CH_MAIN,), jnp.int32),
        pltpu.VMEM((CH_MAIN,), jnp.int32),
        pltpu.VMEM((CH_MAIN,), jnp.float32),
        pltpu.VMEM((TBL,), jnp.float32),
        pltpu.VMEM((SEG_PAD,), jnp.float32),
        pltpu.VMEM((SEG_PAD,), jnp.float32),
        pltpu.VMEM((NTILES, SEG_BLK), jnp.float32),
        pltpu.VMEM((NTILES, SEG_BLK), jnp.float32),
        pltpu.VMEM((SEG_PAD,), jnp.float32),
        pltpu.VMEM_SHARED((NTILES, SEG_PAD), jnp.float32),
        pltpu.VMEM_SHARED((NTILES, SEG_PAD), jnp.float32),
        pltpu.VMEM_SHARED((SEG_PAD,), jnp.float32),
        pltpu.SemaphoreType.DMA,
        pltpu.SemaphoreType.DMA,
    ],
)
def _sc_multiplier(an_hbm, bid_hbm, fr_hbm, out_hbm,
                   an_v, bid_v, raw_v, tbl_v, acc_sum, acc_cnt,
                   red_sum, red_cnt, inv_v, sh_sum, sh_cnt, sh_inv,
                   sem_a, sem_b):
    w = lax.axis_index("s")
    base = w * CH_MAIN
    nelem = jnp.where(w < 15, CH_MAIN, CH_TAIL)

    cp_a = pltpu.async_copy(an_hbm.at[pl.ds(base, CH_TAIL)],
                            an_v.at[pl.ds(0, CH_TAIL)], sem_a)
    cp_b = pltpu.async_copy(bid_hbm.at[pl.ds(base, CH_TAIL)],
                            bid_v.at[pl.ds(0, CH_TAIL)], sem_b)
    pltpu.sync_copy(fr_hbm, tbl_v)

    eps = jnp.float32(1e-8)
    for j in range(TBL // L):
        f = tbl_v[pl.ds(j * L, L)]
        tbl_v[pl.ds(j * L, L)] = 1.0 / (f + eps)

    zero16 = jnp.zeros((L,), jnp.float32)
    for j in range(SEG_PAD // L):
        acc_sum[pl.ds(j * L, L)] = zero16
        acc_cnt[pl.ds(j * L, L)] = zero16

    @pl.when(w < 15)
    def _():
        pltpu.sync_copy(an_hbm.at[pl.ds(base + CH_TAIL, CH_MAIN - CH_TAIL)],
                        an_v.at[pl.ds(CH_TAIL, CH_MAIN - CH_TAIL)])
        pltpu.sync_copy(bid_hbm.at[pl.ds(base + CH_TAIL, CH_MAIN - CH_TAIL)],
                        bid_v.at[pl.ds(CH_TAIL, CH_MAIN - CH_TAIL)])

    cp_a.wait()
    cp_b.wait()

    ones16 = jnp.ones((L,), jnp.float32)

    @plsc.parallel_loop(0, nelem, step=L, unroll=UNROLL)
    def _(i):
        a = an_v[pl.ds(i, L)]
        b = bid_v[pl.ds(i, L)]
        r = plsc.load_gather(tbl_v, [a])
        raw_v[pl.ds(i, L)] = r
        plsc.addupdate_scatter(acc_sum, [b], r)
        plsc.addupdate_scatter(acc_cnt, [b], ones16)

    pltpu.sync_copy(acc_sum, sh_sum.at[w])
    pltpu.sync_copy(acc_cnt, sh_cnt.at[w])
    plsc.subcore_barrier()

    @pl.when(w < 8)
    def _():
        seg0 = w * SEG_BLK
        pltpu.sync_copy(sh_sum.at[:, pl.ds(seg0, SEG_BLK)], red_sum)
        pltpu.sync_copy(sh_cnt.at[:, pl.ds(seg0, SEG_BLK)], red_cnt)
        for sub in range(SEG_BLK // L):
            s = jnp.zeros((L,), jnp.float32)
            c = jnp.zeros((L,), jnp.float32)
            for t in range(NTILES):
                s = s + red_sum[t, pl.ds(sub * L, L)]
                c = c + red_cnt[t, pl.ds(sub * L, L)]
            inv_v[pl.ds(sub * L, L)] = jnp.maximum(c, 1.0) / s
        pltpu.sync_copy(inv_v.at[pl.ds(0, SEG_BLK)],
                        sh_inv.at[pl.ds(seg0, SEG_BLK)])

    plsc.subcore_barrier()
    pltpu.sync_copy(sh_inv, inv_v)

    @plsc.parallel_loop(0, nelem, step=L, unroll=UNROLL)
    def _(i):
        b = bid_v[pl.ds(i, L)]
        r = raw_v[pl.ds(i, L)]
        m = plsc.load_gather(inv_v, [b])
        raw_v[pl.ds(i, L)] = r * m

    pltpu.sync_copy(raw_v.at[pl.ds(0, CH_TAIL)],
                    out_hbm.at[pl.ds(base, CH_TAIL)])

    @pl.when(w < 15)
    def _():
        pltpu.sync_copy(raw_v.at[pl.ds(CH_TAIL, CH_MAIN - CH_TAIL)],
                        out_hbm.at[pl.ds(base + CH_TAIL, CH_MAIN - CH_TAIL)])


def kernel(atomic_numbers, batch_idx, freq_ratios):
    an = atomic_numbers.astype(jnp.int32)
    bid = batch_idx.astype(jnp.int32)
    fr = jnp.pad(freq_ratios, (0, TBL - ATOM_TYPES))
    return _sc_multiplier(an, bid, fr)
